# baseline (speedup 1.0000x reference)
import jax
import jax.numpy as jnp
from jax import lax
from jax.experimental import pallas as pl
from jax.experimental.pallas import tpu as pltpu
from jax.experimental.pallas import tpu_sc as plsc

USER_SIZE = 5000
ITEM_SIZE = 5000
N_NODES = USER_SIZE + ITEM_SIZE
H = 256
HH = 128
E = 160000
NC = 2
NS = 16
NP = 10240
PR = NP // NS
FBLK = 64
HBLK = 80
EROWS = 1280
SROWS = EROWS // NS
GRP = 40
IB = 16
NGRP_N = SROWS // IB

_MESH = plsc.VectorSubcoreMesh(
    core_axis_name="c", subcore_axis_name="s", num_cores=NC, num_subcores=NS)

_F32 = jnp.float32


def _rsqrt16(v):
    i = lax.bitcast_convert_type(v, jnp.int32)
    i = 0x5F3759DF - (i >> 1)
    y = lax.bitcast_convert_type(i, _F32)
    for _ in range(3):
        y = y * (1.5 - 0.5 * v * y * y)
    return y


def _fill2d(buf, rows, val):
    vec = jnp.full((16,), val, _F32)

    def body(r, carry):
        for j in range(8):
            buf[r, pl.ds(j * 16, 16)] = vec
        return carry

    lax.fori_loop(0, rows, body, 0)


def _sget(buf, i):
    return buf[pl.ds(i, 16)][0]


def _lr_copy(c, left_thunk, right_thunk):
    @pl.when(c == 0)
    def _():
        left_thunk()

    @pl.when(c == 1)
    def _():
        right_thunk()


def _norms_kernel(src2d, dst2d, xl, xr, norm_out, norm_in, h0l, h0r,
                  dego_s, degi_s, ones_v, sbufs, dbufs, degbuf, nobuf, nibuf,
                  xbuf, zbuf, csem):
    c = lax.axis_index("c")
    s = lax.axis_index("s")
    off = s * PR

    def zb(r, carry):
        zbuf[pl.ds(r * 16, 16)] = jnp.zeros((16,), _F32)
        return carry
    lax.fori_loop(0, PR // 16, zb, 0)
    pltpu.sync_copy(zbuf.at[pl.ds(0, PR)], dego_s.at[pl.ds(off, PR)])
    pltpu.sync_copy(zbuf.at[pl.ds(0, PR)], degi_s.at[pl.ds(off, PR)])
    for j in range(8):
        ones_v[pl.ds(j * 16, 16)] = jnp.ones((16,), _F32)
    plsc.subcore_barrier()

    def _cwait():
        pltpu.make_async_copy(
            ones_v, dego_s.at[pl.ds(0, 128)], csem).wait()

    def cgrp(g, carry):
        base = s * SROWS + g * IB
        pltpu.sync_copy(src2d.at[pl.ds(base, IB)], sbufs)
        pltpu.sync_copy(dst2d.at[pl.ds(base, IB)], dbufs)

        def cbody(i, carry2):
            pltpu.async_copy(ones_v, dego_s.at[sbufs.at[i]], csem, add=True)
            pltpu.async_copy(ones_v, degi_s.at[dbufs.at[i]], csem, add=True)

            @pl.when(i >= 2)
            def _():
                _cwait()
                _cwait()
            return carry2

        lax.fori_loop(0, IB, cbody, 0)
        for _ in range(4):
            _cwait()
        return carry

    lax.fori_loop(0, NGRP_N, cgrp, 0)
    plsc.subcore_barrier()

    pltpu.sync_copy(dego_s.at[pl.ds(off, PR)], degbuf)

    def nb_out(j, carry):
        v = jnp.maximum(degbuf[pl.ds(j * 16, 16)], 1.0)
        nobuf[pl.ds(j * 16, 16)] = _rsqrt16(v)
        return carry
    lax.fori_loop(0, PR // 16, nb_out, 0)

    pltpu.sync_copy(degi_s.at[pl.ds(off, PR)], degbuf)

    def nb_in(j, carry):
        v = jnp.maximum(degbuf[pl.ds(j * 16, 16)], 1.0)
        nibuf[pl.ds(j * 16, 16)] = _rsqrt16(v)
        return carry
    lax.fori_loop(0, PR // 16, nb_in, 0)

    @pl.when(c == 0)
    def _():
        pltpu.sync_copy(nobuf.at[pl.ds(0, PR)], norm_out.at[pl.ds(off, PR)])
        pltpu.sync_copy(nibuf.at[pl.ds(0, PR)], norm_in.at[pl.ds(off, PR)])

    def hblk(k, carry):
        r0 = off + k * HBLK
        _lr_copy(c,
                 lambda: pltpu.sync_copy(xl.at[pl.ds(r0, HBLK)], xbuf),
                 lambda: pltpu.sync_copy(xr.at[pl.ds(r0, HBLK)], xbuf))

        def srow(r, carry2):
            sc = _sget(nobuf, k * HBLK + r)
            for j in range(8):
                sl = pl.ds(j * 16, 16)
                xbuf[r, sl] = xbuf[r, sl] * sc
            return carry2
        lax.fori_loop(0, HBLK, srow, 0)
        _lr_copy(c,
                 lambda: pltpu.sync_copy(xbuf, h0l.at[pl.ds(r0, HBLK)]),
                 lambda: pltpu.sync_copy(xbuf, h0r.at[pl.ds(r0, HBLK)]))
        return carry

    lax.fori_loop(0, PR // HBLK, hblk, 0)


def _aggregate(c, s, hl, hr, src2d, dst2d, acc_s, sbufs, dbufs, rb0, rb1,
               g0, g1, s0, s1):
    _fill2d(rb0, 128, 0.0)

    def zb(k, carry):
        pltpu.sync_copy(rb0, acc_s.at[pl.ds(s * PR + k * 128, 128)])
        return carry
    lax.fori_loop(0, PR // 128, zb, 0)
    plsc.subcore_barrier()

    rbufs = (rb0, rb1)
    gsems = (g0, g1)
    ssems = (s0, s1)

    def _gather(lr, b):
        _lr_copy(c,
                 lambda: pltpu.async_copy(hl.at[sbufs.at[lr]], rbufs[b],
                                          gsems[b]),
                 lambda: pltpu.async_copy(hr.at[sbufs.at[lr]], rbufs[b],
                                          gsems[b]))

    def _gwait(b):
        pltpu.make_async_copy(hl.at[pl.ds(0, 128)], rbufs[b],
                              gsems[b]).wait()

    def _scat(lr, b):
        pltpu.async_copy(rbufs[b], acc_s.at[dbufs.at[lr]], ssems[b],
                         add=True)

    def _swait(b):
        pltpu.make_async_copy(rbufs[b], acc_s.at[pl.ds(0, 128)],
                              ssems[b]).wait()

    def ebody(r, carry):
        pltpu.sync_copy(src2d.at[pl.ds(r, 1)], sbufs)
        pltpu.sync_copy(dst2d.at[pl.ds(r, 1)], dbufs)
        _lr_copy(c,
                 lambda: pltpu.sync_copy(hl.at[sbufs.at[0]], rb0),
                 lambda: pltpu.sync_copy(hr.at[sbufs.at[0]], rb0))
        pltpu.sync_copy(rb0, acc_s.at[dbufs.at[0]], add=True)
        return carry

    lax.fori_loop(s * SROWS, (s + 1) * SROWS, ebody, 0)
    plsc.subcore_barrier()


def _layer1_kernel(hl, hr, src2d, dst2d, norm_in, norm_out,
                   e1l, e1r, h1l, h1r,
                   acc_s, sbufs, dbufs, rb0, rb1, nibuf, nobuf,
                   g0, g1, s0, s1):
    c = lax.axis_index("c")
    s = lax.axis_index("s")
    _aggregate(c, s, hl, hr, src2d, dst2d, acc_s, sbufs, dbufs, rb0, rb1,
               g0, g1, s0, s1)

    off = s * PR
    pltpu.sync_copy(norm_in.at[pl.ds(off, PR)], nibuf.at[pl.ds(0, PR)])
    pltpu.sync_copy(norm_out.at[pl.ds(off, PR)], nobuf.at[pl.ds(0, PR)])

    def fblk(k, carry):
        r0 = off + k * FBLK
        pltpu.sync_copy(acc_s.at[pl.ds(r0, FBLK)], rb0.at[pl.ds(0, FBLK)])

        def srow(r, carry2):
            ni = _sget(nibuf, k * FBLK + r)
            no = _sget(nobuf, k * FBLK + r)
            for j in range(8):
                sl = pl.ds(j * 16, 16)
                e = rb0[r, sl] * ni
                rb1[r, sl] = e
                rb0[r, sl] = e * no
            return carry2
        lax.fori_loop(0, FBLK, srow, 0)

        _lr_copy(c,
                 lambda: pltpu.sync_copy(rb1.at[pl.ds(0, FBLK)],
                                         e1l.at[pl.ds(r0, FBLK)]),
                 lambda: pltpu.sync_copy(rb1.at[pl.ds(0, FBLK)],
                                         e1r.at[pl.ds(r0, FBLK)]))
        _lr_copy(c,
                 lambda: pltpu.sync_copy(rb0.at[pl.ds(0, FBLK)],
                                         h1l.at[pl.ds(r0, FBLK)]),
                 lambda: pltpu.sync_copy(rb0.at[pl.ds(0, FBLK)],
                                         h1r.at[pl.ds(r0, FBLK)]))
        return carry

    lax.fori_loop(0, PR // FBLK, fblk, 0)


def _layer2_kernel(hl, hr, src2d, dst2d, norm_in, x0l, x0r, e1l, e1r,
                   outl, outr,
                   acc_s, sbufs, dbufs, rb0, rb1, nibuf,
                   g0, g1, s0, s1):
    c = lax.axis_index("c")
    s = lax.axis_index("s")
    _aggregate(c, s, hl, hr, src2d, dst2d, acc_s, sbufs, dbufs, rb0, rb1,
               g0, g1, s0, s1)

    off = s * PR
    pltpu.sync_copy(norm_in.at[pl.ds(off, PR)], nibuf.at[pl.ds(0, PR)])

    def fblk(k, carry):
        r0 = off + k * FBLK
        pltpu.sync_copy(acc_s.at[pl.ds(r0, FBLK)], rb0.at[pl.ds(0, FBLK)])
        _lr_copy(c,
                 lambda: pltpu.sync_copy(x0l.at[pl.ds(r0, FBLK)],
                                         rb0.at[pl.ds(FBLK, FBLK)]),
                 lambda: pltpu.sync_copy(x0r.at[pl.ds(r0, FBLK)],
                                         rb0.at[pl.ds(FBLK, FBLK)]))
        _lr_copy(c,
                 lambda: pltpu.sync_copy(e1l.at[pl.ds(r0, FBLK)],
                                         rb1.at[pl.ds(0, FBLK)]),
                 lambda: pltpu.sync_copy(e1r.at[pl.ds(r0, FBLK)],
                                         rb1.at[pl.ds(0, FBLK)]))

        def rbody(r, carry2):
            ni = _sget(nibuf, k * FBLK + r)
            for j in range(8):
                sl = pl.ds(j * 16, 16)
                rb1[FBLK + r, sl] = (rb0[FBLK + r, sl]
                                     + 0.5 * rb1[r, sl]
                                     + (ni * (1.0 / 3.0)) * rb0[r, sl])
            return carry2
        lax.fori_loop(0, FBLK, rbody, 0)

        _lr_copy(c,
                 lambda: pltpu.sync_copy(rb1.at[pl.ds(FBLK, FBLK)],
                                         outl.at[pl.ds(r0, FBLK)]),
                 lambda: pltpu.sync_copy(rb1.at[pl.ds(FBLK, FBLK)],
                                         outr.at[pl.ds(r0, FBLK)]))
        return carry

    lax.fori_loop(0, PR // FBLK, fblk, 0)


def _sds(shape, dtype=_F32):
    return jax.ShapeDtypeStruct(shape, dtype)


_IDXBUF = pltpu.VMEM((1, 128), jnp.int32)
_ROWBUF = pltpu.VMEM((128, HH), _F32)
_DMA = pltpu.SemaphoreType.DMA

_norms_call = pl.kernel(
    _norms_kernel,
    out_type=(_sds((NP,)), _sds((NP,)), _sds((NP, HH)), _sds((NP, HH))),
    mesh=_MESH,
    scratch_types=[
        pltpu.VMEM_SHARED((NP,), _F32),
        pltpu.VMEM_SHARED((NP,), _F32),
        pltpu.VMEM((128,), _F32),
        pltpu.VMEM((IB, 128), jnp.int32),
        pltpu.VMEM((IB, 128), jnp.int32),
        pltpu.VMEM((PR,), _F32),
        pltpu.VMEM((PR + 16,), _F32),
        pltpu.VMEM((PR + 16,), _F32),
        pltpu.VMEM((HBLK, HH), _F32),
        pltpu.VMEM((PR,), _F32),
        _DMA,
    ],
    name="gcn_norms_h0",
)

_layer1_call = pl.kernel(
    _layer1_kernel,
    out_type=(_sds((NP, HH)), _sds((NP, HH)), _sds((NP, HH)), _sds((NP, HH))),
    mesh=_MESH,
    scratch_types=[
        pltpu.VMEM_SHARED((NP, HH), _F32),
        _IDXBUF, _IDXBUF,
        _ROWBUF, _ROWBUF,
        pltpu.VMEM((PR + 16,), _F32),
        pltpu.VMEM((PR + 16,), _F32),
        _DMA, _DMA,
        _DMA, _DMA,
    ],
    name="gcn_layer1",
)

_layer2_call = pl.kernel(
    _layer2_kernel,
    out_type=(_sds((NP, HH)), _sds((NP, HH))),
    mesh=_MESH,
    scratch_types=[
        pltpu.VMEM_SHARED((NP, HH), _F32),
        _IDXBUF, _IDXBUF,
        _ROWBUF, _ROWBUF,
        pltpu.VMEM((PR + 16,), _F32),
        _DMA, _DMA,
        _DMA, _DMA,
    ],
    name="gcn_layer2_final",
)


def kernel(user_embedding, item_embedding, edge_index):
    x0 = jnp.concatenate([user_embedding, item_embedding], axis=0)
    x0p = jnp.zeros((NP, H), _F32).at[:N_NODES].set(x0)
    x0l = x0p[:, :HH]
    x0r = x0p[:, HH:]
    ei = edge_index.astype(jnp.int32)
    pad = jnp.full((2, EROWS * 128 - E), N_NODES, jnp.int32)
    eip = jnp.concatenate([ei, pad], axis=1)
    src2d = eip[0].reshape(EROWS, 128)
    dst2d = eip[1].reshape(EROWS, 128)

    norm_out, norm_in, h0l, h0r = _norms_call(src2d, dst2d, x0l, x0r)
    e1l, e1r, h1l, h1r = _layer1_call(h0l, h0r, src2d, dst2d, norm_in, norm_out)
    outl, outr = _layer2_call(h1l, h1r, src2d, dst2d, norm_in, x0l, x0r,
                              e1l, e1r)

    full = jnp.concatenate([outl[:N_NODES], outr[:N_NODES]], axis=1)
    return full[:USER_SIZE], full[USER_SIZE:]


# speedup vs baseline: 1.5415x; 1.5415x over previous
import jax
import jax.numpy as jnp
from jax import lax
from jax.experimental import pallas as pl
from jax.experimental.pallas import tpu as pltpu
from jax.experimental.pallas import tpu_sc as plsc

USER_SIZE = 5000
ITEM_SIZE = 5000
N_NODES = USER_SIZE + ITEM_SIZE
H = 256
HH = 128
E = 160000
NC = 2
NS = 16
NP = 10240
PR = NP // NS
FBLK = 64
HBLK = 80
EROWS = 1280
SROWS = EROWS // NS
GRP = 40
IB = 16
NGRP_N = SROWS // IB

_MESH = plsc.VectorSubcoreMesh(
    core_axis_name="c", subcore_axis_name="s", num_cores=NC, num_subcores=NS)

_F32 = jnp.float32


def _rsqrt16(v):
    i = lax.bitcast_convert_type(v, jnp.int32)
    i = 0x5F3759DF - (i >> 1)
    y = lax.bitcast_convert_type(i, _F32)
    for _ in range(3):
        y = y * (1.5 - 0.5 * v * y * y)
    return y


def _fill2d(buf, rows, val):
    vec = jnp.full((16,), val, _F32)

    def body(r, carry):
        for j in range(8):
            buf[r, pl.ds(j * 16, 16)] = vec
        return carry

    lax.fori_loop(0, rows, body, 0)


def _sget(buf, i):
    return buf[pl.ds(i, 16)][0]


def _lr_copy(c, left_thunk, right_thunk):
    @pl.when(c == 0)
    def _():
        left_thunk()

    @pl.when(c == 1)
    def _():
        right_thunk()


def _norms_kernel(src2d, dst2d, xl, xr, norm_out, norm_in, h0l, h0r,
                  dego_s, degi_s, ones_v, sbufs, dbufs, degbuf, nobuf, nibuf,
                  xbuf, zbuf, csem):
    c = lax.axis_index("c")
    s = lax.axis_index("s")
    off = s * PR

    def zb(r, carry):
        zbuf[pl.ds(r * 16, 16)] = jnp.zeros((16,), _F32)
        return carry
    lax.fori_loop(0, PR // 16, zb, 0)
    pltpu.sync_copy(zbuf.at[pl.ds(0, PR)], dego_s.at[pl.ds(off, PR)])
    pltpu.sync_copy(zbuf.at[pl.ds(0, PR)], degi_s.at[pl.ds(off, PR)])
    for j in range(8):
        ones_v[pl.ds(j * 16, 16)] = jnp.ones((16,), _F32)
    plsc.subcore_barrier()

    def _cwait():
        pltpu.make_async_copy(
            ones_v, dego_s.at[pl.ds(0, 128)], csem).wait()

    def cgrp(g, carry):
        base = s * SROWS + g * IB
        pltpu.sync_copy(src2d.at[pl.ds(base, IB)], sbufs)
        pltpu.sync_copy(dst2d.at[pl.ds(base, IB)], dbufs)

        def cbody(i, carry2):
            pltpu.async_copy(ones_v, dego_s.at[sbufs.at[i]], csem, add=True)
            pltpu.async_copy(ones_v, degi_s.at[dbufs.at[i]], csem, add=True)

            @pl.when(i >= 2)
            def _():
                _cwait()
                _cwait()
            return carry2

        lax.fori_loop(0, IB, cbody, 0)
        for _ in range(4):
            _cwait()
        return carry

    lax.fori_loop(0, NGRP_N, cgrp, 0)
    plsc.subcore_barrier()

    pltpu.sync_copy(dego_s.at[pl.ds(off, PR)], degbuf)

    def nb_out(j, carry):
        v = jnp.maximum(degbuf[pl.ds(j * 16, 16)], 1.0)
        nobuf[pl.ds(j * 16, 16)] = _rsqrt16(v)
        return carry
    lax.fori_loop(0, PR // 16, nb_out, 0)

    pltpu.sync_copy(degi_s.at[pl.ds(off, PR)], degbuf)

    def nb_in(j, carry):
        v = jnp.maximum(degbuf[pl.ds(j * 16, 16)], 1.0)
        nibuf[pl.ds(j * 16, 16)] = _rsqrt16(v)
        return carry
    lax.fori_loop(0, PR // 16, nb_in, 0)

    @pl.when(c == 0)
    def _():
        pltpu.sync_copy(nobuf.at[pl.ds(0, PR)], norm_out.at[pl.ds(off, PR)])
        pltpu.sync_copy(nibuf.at[pl.ds(0, PR)], norm_in.at[pl.ds(off, PR)])

    def hblk(k, carry):
        r0 = off + k * HBLK
        _lr_copy(c,
                 lambda: pltpu.sync_copy(xl.at[pl.ds(r0, HBLK)], xbuf),
                 lambda: pltpu.sync_copy(xr.at[pl.ds(r0, HBLK)], xbuf))

        def srow(r, carry2):
            sc = _sget(nobuf, k * HBLK + r)
            for j in range(8):
                sl = pl.ds(j * 16, 16)
                xbuf[r, sl] = xbuf[r, sl] * sc
            return carry2
        lax.fori_loop(0, HBLK, srow, 0)
        _lr_copy(c,
                 lambda: pltpu.sync_copy(xbuf, h0l.at[pl.ds(r0, HBLK)]),
                 lambda: pltpu.sync_copy(xbuf, h0r.at[pl.ds(r0, HBLK)]))
        return carry

    lax.fori_loop(0, PR // HBLK, hblk, 0)


def _aggregate(c, s, hl, hr, src2d, dst2d, acc_s, sbufs, dbufs, rb0, rb1,
               g0, g1, s0, s1):
    _fill2d(rb0, 128, 0.0)

    def zb(k, carry):
        pltpu.sync_copy(rb0, acc_s.at[pl.ds(s * PR + k * 128, 128)])
        return carry
    lax.fori_loop(0, PR // 128, zb, 0)
    plsc.subcore_barrier()

    rbufs = (rb0, rb1)
    gsems = (g0, g1)
    ssems = (s0, s1)

    def _gather(lr, b):
        _lr_copy(c,
                 lambda: pltpu.async_copy(hl.at[sbufs.at[lr]], rbufs[b],
                                          gsems[b]),
                 lambda: pltpu.async_copy(hr.at[sbufs.at[lr]], rbufs[b],
                                          gsems[b]))

    def _gwait(b):
        pltpu.make_async_copy(hl.at[pl.ds(0, 128)], rbufs[b],
                              gsems[b]).wait()

    def _scat(lr, b):
        pltpu.async_copy(rbufs[b], acc_s.at[dbufs.at[lr]], ssems[b],
                         add=True)

    def _swait(b):
        pltpu.make_async_copy(rbufs[b], acc_s.at[pl.ds(0, 128)],
                              ssems[b]).wait()

    def ebody(r, carry):
        pltpu.sync_copy(src2d.at[pl.ds(r, 1)], sbufs)
        pltpu.sync_copy(dst2d.at[pl.ds(r, 1)], dbufs)
        _lr_copy(c,
                 lambda: pltpu.sync_copy(hl.at[sbufs.at[0]], rb0),
                 lambda: pltpu.sync_copy(hr.at[sbufs.at[0]], rb0))
        pltpu.sync_copy(rb0, acc_s.at[dbufs.at[0]], add=True)
        return carry

    lax.fori_loop(s * SROWS, (s + 1) * SROWS, ebody, 0)
    plsc.subcore_barrier()


def _layer1_kernel(hl, hr, src2d, dst2d, norm_in, norm_out,
                   e1l, e1r, h1l, h1r,
                   acc_s, sbufs, dbufs, rb0, rb1, nibuf, nobuf,
                   g0, g1, s0, s1):
    c = lax.axis_index("c")
    s = lax.axis_index("s")
    _aggregate(c, s, hl, hr, src2d, dst2d, acc_s, sbufs, dbufs, rb0, rb1,
               g0, g1, s0, s1)

    off = s * PR
    pltpu.sync_copy(norm_in.at[pl.ds(off, PR)], nibuf.at[pl.ds(0, PR)])
    pltpu.sync_copy(norm_out.at[pl.ds(off, PR)], nobuf.at[pl.ds(0, PR)])

    def fblk(k, carry):
        r0 = off + k * FBLK
        pltpu.sync_copy(acc_s.at[pl.ds(r0, FBLK)], rb0.at[pl.ds(0, FBLK)])

        def srow(r, carry2):
            ni = _sget(nibuf, k * FBLK + r)
            no = _sget(nobuf, k * FBLK + r)
            for j in range(8):
                sl = pl.ds(j * 16, 16)
                e = rb0[r, sl] * ni
                rb1[r, sl] = e
                rb0[r, sl] = e * no
            return carry2
        lax.fori_loop(0, FBLK, srow, 0)

        _lr_copy(c,
                 lambda: pltpu.sync_copy(rb1.at[pl.ds(0, FBLK)],
                                         e1l.at[pl.ds(r0, FBLK)]),
                 lambda: pltpu.sync_copy(rb1.at[pl.ds(0, FBLK)],
                                         e1r.at[pl.ds(r0, FBLK)]))
        _lr_copy(c,
                 lambda: pltpu.sync_copy(rb0.at[pl.ds(0, FBLK)],
                                         h1l.at[pl.ds(r0, FBLK)]),
                 lambda: pltpu.sync_copy(rb0.at[pl.ds(0, FBLK)],
                                         h1r.at[pl.ds(r0, FBLK)]))
        return carry

    lax.fori_loop(0, PR // FBLK, fblk, 0)


def _layer2_kernel(hl, hr, src2d, dst2d, norm_in, x0l, x0r, e1l, e1r,
                   outl, outr,
                   acc_s, sbufs, dbufs, rb0, rb1, nibuf,
                   g0, g1, s0, s1):
    c = lax.axis_index("c")
    s = lax.axis_index("s")
    _aggregate(c, s, hl, hr, src2d, dst2d, acc_s, sbufs, dbufs, rb0, rb1,
               g0, g1, s0, s1)

    off = s * PR
    pltpu.sync_copy(norm_in.at[pl.ds(off, PR)], nibuf.at[pl.ds(0, PR)])

    def fblk(k, carry):
        r0 = off + k * FBLK
        pltpu.sync_copy(acc_s.at[pl.ds(r0, FBLK)], rb0.at[pl.ds(0, FBLK)])
        _lr_copy(c,
                 lambda: pltpu.sync_copy(x0l.at[pl.ds(r0, FBLK)],
                                         rb0.at[pl.ds(FBLK, FBLK)]),
                 lambda: pltpu.sync_copy(x0r.at[pl.ds(r0, FBLK)],
                                         rb0.at[pl.ds(FBLK, FBLK)]))
        _lr_copy(c,
                 lambda: pltpu.sync_copy(e1l.at[pl.ds(r0, FBLK)],
                                         rb1.at[pl.ds(0, FBLK)]),
                 lambda: pltpu.sync_copy(e1r.at[pl.ds(r0, FBLK)],
                                         rb1.at[pl.ds(0, FBLK)]))

        def rbody(r, carry2):
            ni = _sget(nibuf, k * FBLK + r)
            for j in range(8):
                sl = pl.ds(j * 16, 16)
                rb1[FBLK + r, sl] = (rb0[FBLK + r, sl]
                                     + 0.5 * rb1[r, sl]
                                     + (ni * (1.0 / 3.0)) * rb0[r, sl])
            return carry2
        lax.fori_loop(0, FBLK, rbody, 0)

        _lr_copy(c,
                 lambda: pltpu.sync_copy(rb1.at[pl.ds(FBLK, FBLK)],
                                         outl.at[pl.ds(r0, FBLK)]),
                 lambda: pltpu.sync_copy(rb1.at[pl.ds(FBLK, FBLK)],
                                         outr.at[pl.ds(r0, FBLK)]))
        return carry

    lax.fori_loop(0, PR // FBLK, fblk, 0)


def _sds(shape, dtype=_F32):
    return jax.ShapeDtypeStruct(shape, dtype)


_IDXBUF = pltpu.VMEM((1, 128), jnp.int32)
_ROWBUF = pltpu.VMEM((128, HH), _F32)
_DMA = pltpu.SemaphoreType.DMA

_norms_call = pl.kernel(
    _norms_kernel,
    out_type=(_sds((NP,)), _sds((NP,)), _sds((NP, HH)), _sds((NP, HH))),
    mesh=_MESH,
    scratch_types=[
        pltpu.VMEM_SHARED((NP,), _F32),
        pltpu.VMEM_SHARED((NP,), _F32),
        pltpu.VMEM((128,), _F32),
        pltpu.VMEM((IB, 128), jnp.int32),
        pltpu.VMEM((IB, 128), jnp.int32),
        pltpu.VMEM((PR,), _F32),
        pltpu.VMEM((PR + 16,), _F32),
        pltpu.VMEM((PR + 16,), _F32),
        pltpu.VMEM((HBLK, HH), _F32),
        pltpu.VMEM((PR,), _F32),
        _DMA,
    ],
    name="gcn_norms_h0",
)

_layer1_call = pl.kernel(
    _layer1_kernel,
    out_type=(_sds((NP, HH)), _sds((NP, HH)), _sds((NP, HH)), _sds((NP, HH))),
    mesh=_MESH,
    scratch_types=[
        pltpu.VMEM_SHARED((NP, HH), _F32),
        _IDXBUF, _IDXBUF,
        _ROWBUF, _ROWBUF,
        pltpu.VMEM((PR + 16,), _F32),
        pltpu.VMEM((PR + 16,), _F32),
        _DMA, _DMA,
        _DMA, _DMA,
    ],
    name="gcn_layer1",
)

_layer2_call = pl.kernel(
    _layer2_kernel,
    out_type=(_sds((NP, HH)), _sds((NP, HH))),
    mesh=_MESH,
    scratch_types=[
        pltpu.VMEM_SHARED((NP, HH), _F32),
        _IDXBUF, _IDXBUF,
        _ROWBUF, _ROWBUF,
        pltpu.VMEM((PR + 16,), _F32),
        _DMA, _DMA,
        _DMA, _DMA,
    ],
    name="gcn_layer2_final",
)


def kernel(user_embedding, item_embedding, edge_index):
    x0 = jnp.concatenate([user_embedding, item_embedding], axis=0)
    x0p = jnp.zeros((NP, H), _F32).at[:N_NODES].set(x0)
    x0l = x0p[:, :HH]
    x0r = x0p[:, HH:]
    ei = edge_index.astype(jnp.int32)
    padv = N_NODES + (jnp.arange(EROWS * 128 - E, dtype=jnp.int32)
                      % (NP - N_NODES))
    eip = jnp.concatenate([ei, jnp.stack([padv, padv])], axis=1)
    src2d = eip[0].reshape(EROWS, 128)
    dst2d = eip[1].reshape(EROWS, 128)

    norm_out, norm_in, h0l, h0r = _norms_call(src2d, dst2d, x0l, x0r)
    e1l, e1r, h1l, h1r = _layer1_call(h0l, h0r, src2d, dst2d, norm_in, norm_out)
    outl, outr = _layer2_call(h1l, h1r, src2d, dst2d, norm_in, x0l, x0r,
                              e1l, e1r)

    full = jnp.concatenate([outl[:N_NODES], outr[:N_NODES]], axis=1)
    return full[:USER_SIZE], full[USER_SIZE:]


# speedup vs baseline: 2.4254x; 1.5734x over previous
import jax
import jax.numpy as jnp
from jax import lax
from jax.experimental import pallas as pl
from jax.experimental.pallas import tpu as pltpu
from jax.experimental.pallas import tpu_sc as plsc

USER_SIZE = 5000
ITEM_SIZE = 5000
N_NODES = USER_SIZE + ITEM_SIZE
H = 256
HH = 128
E = 160000
NC = 2
NS = 16
NP = 10240
PR = NP // NS
FBLK = 64
HBLK = 80
EROWS = 1280
SROWS = EROWS // NS
GRP = 40
IB = 16
NGRP_N = SROWS // IB

_MESH = plsc.VectorSubcoreMesh(
    core_axis_name="c", subcore_axis_name="s", num_cores=NC, num_subcores=NS)

_F32 = jnp.float32


def _rsqrt16(v):
    i = lax.bitcast_convert_type(v, jnp.int32)
    i = 0x5F3759DF - (i >> 1)
    y = lax.bitcast_convert_type(i, _F32)
    for _ in range(3):
        y = y * (1.5 - 0.5 * v * y * y)
    return y


def _fill2d(buf, rows, val):
    vec = jnp.full((16,), val, _F32)

    def body(r, carry):
        for j in range(8):
            buf[r, pl.ds(j * 16, 16)] = vec
        return carry

    lax.fori_loop(0, rows, body, 0)


def _sget(buf, i):
    return buf[pl.ds(i, 16)][0]


def _lr_copy(c, left_thunk, right_thunk):
    @pl.when(c == 0)
    def _():
        left_thunk()

    @pl.when(c == 1)
    def _():
        right_thunk()


def _norms_kernel(src2d, dst2d, xl, xr, norm_out, norm_in, h0l, h0r,
                  dego_s, degi_s, ones_v, sbufs, dbufs, degbuf, nobuf, nibuf,
                  xbuf, zbuf, csem):
    c = lax.axis_index("c")
    s = lax.axis_index("s")
    off = s * PR

    def zb(r, carry):
        zbuf[pl.ds(r * 16, 16)] = jnp.zeros((16,), _F32)
        return carry
    lax.fori_loop(0, PR // 16, zb, 0)
    pltpu.sync_copy(zbuf.at[pl.ds(0, PR)], dego_s.at[pl.ds(off, PR)])
    pltpu.sync_copy(zbuf.at[pl.ds(0, PR)], degi_s.at[pl.ds(off, PR)])
    for j in range(8):
        ones_v[pl.ds(j * 16, 16)] = jnp.ones((16,), _F32)
    plsc.subcore_barrier()

    def _cwait():
        pltpu.make_async_copy(
            ones_v, dego_s.at[pl.ds(0, 128)], csem).wait()

    def cgrp(g, carry):
        base = s * SROWS + g * IB
        pltpu.sync_copy(src2d.at[pl.ds(base, IB)], sbufs)
        pltpu.sync_copy(dst2d.at[pl.ds(base, IB)], dbufs)

        def cbody(i, carry2):
            pltpu.async_copy(ones_v, dego_s.at[sbufs.at[i]], csem, add=True)
            pltpu.async_copy(ones_v, degi_s.at[dbufs.at[i]], csem, add=True)

            @pl.when(i >= 2)
            def _():
                _cwait()
                _cwait()
            return carry2

        lax.fori_loop(0, IB, cbody, 0)
        for _ in range(4):
            _cwait()
        return carry

    lax.fori_loop(0, NGRP_N, cgrp, 0)
    plsc.subcore_barrier()

    pltpu.sync_copy(dego_s.at[pl.ds(off, PR)], degbuf)

    def nb_out(j, carry):
        v = jnp.maximum(degbuf[pl.ds(j * 16, 16)], 1.0)
        nobuf[pl.ds(j * 16, 16)] = _rsqrt16(v)
        return carry
    lax.fori_loop(0, PR // 16, nb_out, 0)

    pltpu.sync_copy(degi_s.at[pl.ds(off, PR)], degbuf)

    def nb_in(j, carry):
        v = jnp.maximum(degbuf[pl.ds(j * 16, 16)], 1.0)
        nibuf[pl.ds(j * 16, 16)] = _rsqrt16(v)
        return carry
    lax.fori_loop(0, PR // 16, nb_in, 0)

    @pl.when(c == 0)
    def _():
        pltpu.sync_copy(nobuf.at[pl.ds(0, PR)], norm_out.at[pl.ds(off, PR)])
        pltpu.sync_copy(nibuf.at[pl.ds(0, PR)], norm_in.at[pl.ds(off, PR)])

    def hblk(k, carry):
        r0 = off + k * HBLK
        _lr_copy(c,
                 lambda: pltpu.sync_copy(xl.at[pl.ds(r0, HBLK)], xbuf),
                 lambda: pltpu.sync_copy(xr.at[pl.ds(r0, HBLK)], xbuf))

        def srow(r, carry2):
            sc = _sget(nobuf, k * HBLK + r)
            for j in range(8):
                sl = pl.ds(j * 16, 16)
                xbuf[r, sl] = xbuf[r, sl] * sc
            return carry2
        lax.fori_loop(0, HBLK, srow, 0)
        _lr_copy(c,
                 lambda: pltpu.sync_copy(xbuf, h0l.at[pl.ds(r0, HBLK)]),
                 lambda: pltpu.sync_copy(xbuf, h0r.at[pl.ds(r0, HBLK)]))
        return carry

    lax.fori_loop(0, PR // HBLK, hblk, 0)


def _aggregate(c, s, hl, hr, src2d, dst2d, acc_s, sbufs, dbufs, rb0, rb1,
               g0, g1, s0, s1):
    _fill2d(rb0, 128, 0.0)

    def zb(k, carry):
        pltpu.sync_copy(rb0, acc_s.at[pl.ds(s * PR + k * 128, 128)])
        return carry
    lax.fori_loop(0, PR // 128, zb, 0)
    plsc.subcore_barrier()

    rbufs = (rb0, rb1)
    gsems = (g0, g1)
    ssems = (s0, s1)

    def _gather(lr, b):
        _lr_copy(c,
                 lambda: pltpu.async_copy(hl.at[sbufs.at[lr]], rbufs[b],
                                          gsems[b]),
                 lambda: pltpu.async_copy(hr.at[sbufs.at[lr]], rbufs[b],
                                          gsems[b]))

    def _gwait(b):
        pltpu.make_async_copy(hl.at[pl.ds(0, 128)], rbufs[b],
                              gsems[b]).wait()

    def _scat(lr, b):
        pltpu.async_copy(rbufs[b], acc_s.at[dbufs.at[lr]], ssems[b],
                         add=True)

    def _swait(b):
        pltpu.make_async_copy(rbufs[b], acc_s.at[pl.ds(0, 128)],
                              ssems[b]).wait()

    def egrp(g, carry):
        base = s * SROWS + g * GRP
        pltpu.sync_copy(src2d.at[pl.ds(base, GRP)], sbufs)
        pltpu.sync_copy(dst2d.at[pl.ds(base, GRP)], dbufs)

        _gather(0, 0)
        _gwait(0)
        _gather(1, 1)
        _scat(0, 0)

        def ebody(i, carry2):
            _gwait(1)
            _swait(0)
            _gather(2 * i + 2, 0)
            _scat(2 * i + 1, 1)
            _gwait(0)
            _swait(1)
            _gather(2 * i + 3, 1)
            _scat(2 * i + 2, 0)
            return carry2

        lax.fori_loop(0, GRP // 2 - 1, ebody, 0)
        _gwait(1)
        _swait(0)
        _scat(GRP - 1, 1)
        _swait(1)
        return carry

    lax.fori_loop(0, SROWS // GRP, egrp, 0)
    plsc.subcore_barrier()


def _layer1_kernel(hl, hr, src2d, dst2d, norm_in, norm_out,
                   e1l, e1r, h1l, h1r,
                   acc_s, sbufs, dbufs, rb0, rb1, nibuf, nobuf,
                   g0, g1, s0, s1):
    c = lax.axis_index("c")
    s = lax.axis_index("s")
    _aggregate(c, s, hl, hr, src2d, dst2d, acc_s, sbufs, dbufs, rb0, rb1,
               g0, g1, s0, s1)

    off = s * PR
    pltpu.sync_copy(norm_in.at[pl.ds(off, PR)], nibuf.at[pl.ds(0, PR)])
    pltpu.sync_copy(norm_out.at[pl.ds(off, PR)], nobuf.at[pl.ds(0, PR)])

    def fblk(k, carry):
        r0 = off + k * FBLK
        pltpu.sync_copy(acc_s.at[pl.ds(r0, FBLK)], rb0.at[pl.ds(0, FBLK)])

        def srow(r, carry2):
            ni = _sget(nibuf, k * FBLK + r)
            no = _sget(nobuf, k * FBLK + r)
            for j in range(8):
                sl = pl.ds(j * 16, 16)
                e = rb0[r, sl] * ni
                rb1[r, sl] = e
                rb0[r, sl] = e * no
            return carry2
        lax.fori_loop(0, FBLK, srow, 0)

        _lr_copy(c,
                 lambda: pltpu.sync_copy(rb1.at[pl.ds(0, FBLK)],
                                         e1l.at[pl.ds(r0, FBLK)]),
                 lambda: pltpu.sync_copy(rb1.at[pl.ds(0, FBLK)],
                                         e1r.at[pl.ds(r0, FBLK)]))
        _lr_copy(c,
                 lambda: pltpu.sync_copy(rb0.at[pl.ds(0, FBLK)],
                                         h1l.at[pl.ds(r0, FBLK)]),
                 lambda: pltpu.sync_copy(rb0.at[pl.ds(0, FBLK)],
                                         h1r.at[pl.ds(r0, FBLK)]))
        return carry

    lax.fori_loop(0, PR // FBLK, fblk, 0)


def _layer2_kernel(hl, hr, src2d, dst2d, norm_in, x0l, x0r, e1l, e1r,
                   outl, outr,
                   acc_s, sbufs, dbufs, rb0, rb1, nibuf,
                   g0, g1, s0, s1):
    c = lax.axis_index("c")
    s = lax.axis_index("s")
    _aggregate(c, s, hl, hr, src2d, dst2d, acc_s, sbufs, dbufs, rb0, rb1,
               g0, g1, s0, s1)

    off = s * PR
    pltpu.sync_copy(norm_in.at[pl.ds(off, PR)], nibuf.at[pl.ds(0, PR)])

    def fblk(k, carry):
        r0 = off + k * FBLK
        pltpu.sync_copy(acc_s.at[pl.ds(r0, FBLK)], rb0.at[pl.ds(0, FBLK)])
        _lr_copy(c,
                 lambda: pltpu.sync_copy(x0l.at[pl.ds(r0, FBLK)],
                                         rb0.at[pl.ds(FBLK, FBLK)]),
                 lambda: pltpu.sync_copy(x0r.at[pl.ds(r0, FBLK)],
                                         rb0.at[pl.ds(FBLK, FBLK)]))
        _lr_copy(c,
                 lambda: pltpu.sync_copy(e1l.at[pl.ds(r0, FBLK)],
                                         rb1.at[pl.ds(0, FBLK)]),
                 lambda: pltpu.sync_copy(e1r.at[pl.ds(r0, FBLK)],
                                         rb1.at[pl.ds(0, FBLK)]))

        def rbody(r, carry2):
            ni = _sget(nibuf, k * FBLK + r)
            for j in range(8):
                sl = pl.ds(j * 16, 16)
                rb1[FBLK + r, sl] = (rb0[FBLK + r, sl]
                                     + 0.5 * rb1[r, sl]
                                     + (ni * (1.0 / 3.0)) * rb0[r, sl])
            return carry2
        lax.fori_loop(0, FBLK, rbody, 0)

        _lr_copy(c,
                 lambda: pltpu.sync_copy(rb1.at[pl.ds(FBLK, FBLK)],
                                         outl.at[pl.ds(r0, FBLK)]),
                 lambda: pltpu.sync_copy(rb1.at[pl.ds(FBLK, FBLK)],
                                         outr.at[pl.ds(r0, FBLK)]))
        return carry

    lax.fori_loop(0, PR // FBLK, fblk, 0)


def _sds(shape, dtype=_F32):
    return jax.ShapeDtypeStruct(shape, dtype)


_IDXBUF = pltpu.VMEM((GRP, 128), jnp.int32)
_ROWBUF = pltpu.VMEM((128, HH), _F32)
_DMA = pltpu.SemaphoreType.DMA

_norms_call = pl.kernel(
    _norms_kernel,
    out_type=(_sds((NP,)), _sds((NP,)), _sds((NP, HH)), _sds((NP, HH))),
    mesh=_MESH,
    scratch_types=[
        pltpu.VMEM_SHARED((NP,), _F32),
        pltpu.VMEM_SHARED((NP,), _F32),
        pltpu.VMEM((128,), _F32),
        pltpu.VMEM((IB, 128), jnp.int32),
        pltpu.VMEM((IB, 128), jnp.int32),
        pltpu.VMEM((PR,), _F32),
        pltpu.VMEM((PR + 16,), _F32),
        pltpu.VMEM((PR + 16,), _F32),
        pltpu.VMEM((HBLK, HH), _F32),
        pltpu.VMEM((PR,), _F32),
        _DMA,
    ],
    name="gcn_norms_h0",
)

_layer1_call = pl.kernel(
    _layer1_kernel,
    out_type=(_sds((NP, HH)), _sds((NP, HH)), _sds((NP, HH)), _sds((NP, HH))),
    mesh=_MESH,
    scratch_types=[
        pltpu.VMEM_SHARED((NP, HH), _F32),
        _IDXBUF, _IDXBUF,
        _ROWBUF, _ROWBUF,
        pltpu.VMEM((PR + 16,), _F32),
        pltpu.VMEM((PR + 16,), _F32),
        _DMA, _DMA,
        _DMA, _DMA,
    ],
    name="gcn_layer1",
)

_layer2_call = pl.kernel(
    _layer2_kernel,
    out_type=(_sds((NP, HH)), _sds((NP, HH))),
    mesh=_MESH,
    scratch_types=[
        pltpu.VMEM_SHARED((NP, HH), _F32),
        _IDXBUF, _IDXBUF,
        _ROWBUF, _ROWBUF,
        pltpu.VMEM((PR + 16,), _F32),
        _DMA, _DMA,
        _DMA, _DMA,
    ],
    name="gcn_layer2_final",
)


def kernel(user_embedding, item_embedding, edge_index):
    x0 = jnp.concatenate([user_embedding, item_embedding], axis=0)
    x0p = jnp.zeros((NP, H), _F32).at[:N_NODES].set(x0)
    x0l = x0p[:, :HH]
    x0r = x0p[:, HH:]
    ei = edge_index.astype(jnp.int32)
    padv = N_NODES + (jnp.arange(EROWS * 128 - E, dtype=jnp.int32)
                      % (NP - N_NODES))
    eip = jnp.concatenate([ei, jnp.stack([padv, padv])], axis=1)
    src2d = eip[0].reshape(EROWS, 128)
    dst2d = eip[1].reshape(EROWS, 128)

    norm_out, norm_in, h0l, h0r = _norms_call(src2d, dst2d, x0l, x0r)
    e1l, e1r, h1l, h1r = _layer1_call(h0l, h0r, src2d, dst2d, norm_in, norm_out)
    outl, outr = _layer2_call(h1l, h1r, src2d, dst2d, norm_in, x0l, x0r,
                              e1l, e1r)

    full = jnp.concatenate([outl[:N_NODES], outr[:N_NODES]], axis=1)
    return full[:USER_SIZE], full[USER_SIZE:]
